# baseline (speedup 1.0000x reference)
import jax
import jax.numpy as jnp
from jax import lax
from jax.experimental import pallas as pl
from jax.experimental.pallas import tpu as pltpu
from jax.experimental.pallas import tpu_sc as plsc

N = 10000
E = 320000
F = 128
K = 8
FD = 16
ITERS = 4

NC = 2
NS = 16
CH = 40
CHUNKS = E // CH
CPC = CHUNKS // NC
NJ = CPC // NS
RPT = 624
SP = 48
NSP = RPT // SP
TBASE = RPT * NS
NG = 1280
GPT = NG // NS

BL = 1000

_GDN = lax.GatherDimensionNumbers(
    offset_dims=(), collapsed_slice_dims=(0,), start_index_map=(0,))


def _perm(v, idx):
    return lax.gather(v, idx[:, None], _GDN, slice_sizes=(1,),
                      mode=lax.GatherScatterMode.PROMISE_IN_BOUNDS)


def _tc_init_body(x_ref, w_ref, b_ref, bm_ref, o_ref):
    h = jnp.dot(x_ref[...], w_ref[...], preferred_element_type=jnp.float32)
    h = h + b_ref[...]
    h = jnp.where(h >= 0.0, h, 0.01 * h)
    cs = jnp.dot(h * h, bm_ref[...], preferred_element_type=jnp.float32)
    o_ref[...] = h * lax.rsqrt(cs)


_tc_init = pl.pallas_call(
    _tc_init_body,
    grid=(N // BL,),
    in_specs=[
        pl.BlockSpec((BL, F), lambda i: (i, 0)),
        pl.BlockSpec((F, F), lambda i: (0, 0)),
        pl.BlockSpec((1, F), lambda i: (0, 0)),
        pl.BlockSpec((F, F), lambda i: (0, 0)),
    ],
    out_specs=pl.BlockSpec((BL, F), lambda i: (i, 0)),
    out_shape=jax.ShapeDtypeStruct((N, F), jnp.float32),
)


def _tc_node_body(num_ref, s_ref, hn_ref, p_ref, bm_ref, o_ref):
    nsum = num_ref[0] + num_ref[1]
    ssum = s_ref[0] + s_ref[1]
    sb = jnp.dot(ssum, p_ref[...], preferred_element_type=jnp.float32)
    sb = jnp.where(sb > 0.0, sb, 1.0)
    na = nsum / sb + hn_ref[...]
    cs = jnp.dot(na * na, bm_ref[...], preferred_element_type=jnp.float32)
    o_ref[...] = na * lax.rsqrt(cs)


_tc_node = pl.pallas_call(
    _tc_node_body,
    grid=(N // BL,),
    in_specs=[
        pl.BlockSpec((NC, BL, F), lambda i: (0, i, 0)),
        pl.BlockSpec((NC, BL, FD), lambda i: (0, i, 0)),
        pl.BlockSpec((BL, F), lambda i: (i, 0)),
        pl.BlockSpec((FD, F), lambda i: (0, 0)),
        pl.BlockSpec((F, F), lambda i: (0, 0)),
    ],
    out_specs=pl.BlockSpec((BL, F), lambda i: (i, 0)),
    out_shape=jax.ShapeDtypeStruct((N, F), jnp.float32),
)


def _sc_edge_body(hd, hn, src, dst, znum, num2, s2,
                  num_s, s_g,
                  isrc0, isrcp0, idst0, idx80, hd0, hn0,
                  isrc1, isrcp1, idst1, idx81, hd1, hn1,
                  s_v, semA0, semB0, semA1, semB1):
    core = lax.axis_index("c")
    sub = lax.axis_index("s")
    r0 = sub * RPT
    g0 = sub * GPT

    pltpu.sync_copy(znum.at[pl.ds(0, CH)], s_v)

    def zpiece(t, czp):
        pltpu.sync_copy(s_v.at[pl.ds(0, CH)], num_s.at[pl.ds(r0 + CH * t, CH)])
        return czp

    lax.fori_loop(0, RPT // CH, zpiece, 0)
    pltpu.sync_copy(s_v.at[pl.ds(0, 24)], num_s.at[pl.ds(r0 + (RPT // CH) * CH, 24)])
    pltpu.sync_copy(s_v.at[pl.ds(0, FD)], num_s.at[pl.ds(TBASE, FD)])
    pltpu.sync_copy(s_v, s_g.at[pl.ds(g0, CH)])
    pltpu.sync_copy(s_v, s_g.at[pl.ds(g0 + CH, CH)])
    plsc.subcore_barrier()

    iot = lax.iota(jnp.int32, FD)
    perms = [jnp.bitwise_xor(iot, sh) for sh in (8, 4, 2, 1)]
    zrow = jnp.zeros((FD,), jnp.float32)
    cbase = core * CPC + sub
    clast = (core + 1) * CPC - NS + sub

    def load_idx(cc, isrc_b, isrcp_b, idst_b, idx8_b):
        eb = cc * CH
        pltpu.sync_copy(src.at[pl.ds(eb, CH)], isrc_b)
        pltpu.sync_copy(src.at[pl.ds(eb, CH)], isrcp_b.at[pl.ds(0, CH)])
        pltpu.sync_copy(dst.at[pl.ds(eb, CH)], idst_b)
        for off in (0, 16, CH - FD):
            idx8_b[pl.ds(off, FD)] = lax.shift_right_logical(
                isrc_b[pl.ds(off, FD)], 3)

    def fire(isrc_b, idst_b, hd_b, hn_b, semA, semB):
        pltpu.async_copy(hd.at[isrc_b], hd_b, semA)
        pltpu.async_copy(hn.at[idst_b], hn_b, semB)

    def wait(hd_b, hn_b, semA, semB):
        pltpu.make_async_copy(hd, hd_b, semA).wait()
        pltpu.make_async_copy(hn, hn_b, semB).wait()

    def compute_scatter(isrc_b, isrcp_b, idx8_b, hd_b, hn_b):
        def edge(i2, carry2):
            for half in range(2):
                i = i2 * 2 + half
                exrow = zrow
                for k in range(K):
                    a = hd_b[i, pl.ds(k * FD, FD)]
                    b = hn_b[i, pl.ds(k * FD, FD)]
                    p = a * b
                    for pm in perms:
                        p = p + _perm(p, pm)
                    ek = jnp.exp(p)
                    hn_b[i, pl.ds(k * FD, FD)] = b * ek
                    exrow = jnp.where(iot == k, ek, exrow)
                sv = isrcp_b[pl.ds(i, FD)]
                boff = jnp.bitwise_and(sv[0], 7)
                for blk in range(8):
                    s_v[i, pl.ds(blk * FD, FD)] = jnp.where(boff == blk, exrow, zrow)
            return carry2

        lax.fori_loop(0, CH // 2, edge, 0)
        pltpu.sync_copy(hn_b, num_s.at[isrc_b], add=True)
        pltpu.sync_copy(s_v, s_g.at[idx8_b], add=True)

    load_idx(cbase, isrc0, isrcp0, idst0, idx80)
    fire(isrc0, idst0, hd0, hn0, semA0, semB0)
    load_idx(cbase + NS, isrc1, isrcp1, idst1, idx81)

    def pair(j2, carry):
        j = 2 * j2
        fire(isrc1, idst1, hd1, hn1, semA1, semB1)
        wait(hd0, hn0, semA0, semB0)
        compute_scatter(isrc0, isrcp0, idx80, hd0, hn0)
        load_idx(jnp.minimum(cbase + NS * (j + 2), clast), isrc0, isrcp0, idst0, idx80)
        fire(isrc0, idst0, hd0, hn0, semA0, semB0)
        wait(hd1, hn1, semA1, semB1)
        compute_scatter(isrc1, isrcp1, idx81, hd1, hn1)
        load_idx(jnp.minimum(cbase + NS * (j + 3), clast), isrc1, isrcp1, idst1, idx81)
        return carry

    lax.fori_loop(0, NJ // 2, pair, 0)
    wait(hd0, hn0, semA0, semB0)
    plsc.subcore_barrier()

    def opiece(t, cop):
        ro = r0 + CH * t
        pltpu.sync_copy(num_s.at[pl.ds(ro, CH)], hd0.at[pl.ds(0, CH)])
        pltpu.sync_copy(hd0.at[pl.ds(0, CH)], num2.at[core, pl.ds(ro, CH)])
        return cop

    lax.fori_loop(0, RPT // CH, opiece, 0)
    rt = r0 + (RPT // CH) * CH
    pltpu.sync_copy(num_s.at[pl.ds(rt, 24)], hd0.at[pl.ds(0, 24)])
    pltpu.sync_copy(hd0.at[pl.ds(0, 24)], num2.at[core, pl.ds(rt, 24)])
    pltpu.sync_copy(num_s.at[pl.ds(TBASE, FD)], hd0.at[pl.ds(0, FD)])
    pltpu.sync_copy(hd0.at[pl.ds(0, FD)], num2.at[core, pl.ds(TBASE, FD)])
    pltpu.sync_copy(s_g.at[pl.ds(g0, CH)], hd0.at[pl.ds(0, CH)])
    pltpu.sync_copy(hd0.at[pl.ds(0, CH)], s2.at[core, pl.ds(g0, CH)])
    pltpu.sync_copy(s_g.at[pl.ds(g0 + CH, CH)], hd0.at[pl.ds(0, CH)])
    pltpu.sync_copy(hd0.at[pl.ds(0, CH)], s2.at[core, pl.ds(g0 + CH, CH)])


_sc_edge = pl.kernel(
    _sc_edge_body,
    out_type=(
        jax.ShapeDtypeStruct((NC, N, F), jnp.float32),
        jax.ShapeDtypeStruct((NC, NG, F), jnp.float32),
    ),
    mesh=plsc.VectorSubcoreMesh(
        core_axis_name="c", subcore_axis_name="s", num_cores=NC, num_subcores=NS
    ),
    scratch_types=[
        pltpu.VMEM_SHARED((N, F), jnp.float32),
        pltpu.VMEM_SHARED((NG, F), jnp.float32),
        pltpu.VMEM((CH,), jnp.int32),
        pltpu.VMEM((CH + FD,), jnp.int32),
        pltpu.VMEM((CH,), jnp.int32),
        pltpu.VMEM((CH,), jnp.int32),
        pltpu.VMEM((CH, F), jnp.float32),
        pltpu.VMEM((CH, F), jnp.float32),
        pltpu.VMEM((CH,), jnp.int32),
        pltpu.VMEM((CH + FD,), jnp.int32),
        pltpu.VMEM((CH,), jnp.int32),
        pltpu.VMEM((CH,), jnp.int32),
        pltpu.VMEM((CH, F), jnp.float32),
        pltpu.VMEM((CH, F), jnp.float32),
        pltpu.VMEM((CH, F), jnp.float32),
        pltpu.SemaphoreType.DMA,
        pltpu.SemaphoreType.DMA,
        pltpu.SemaphoreType.DMA,
        pltpu.SemaphoreType.DMA,
    ],
)


@jax.jit
def kernel(x, edge_index, weight, bias):
    src = edge_index[0]
    dst = edge_index[1]
    col = jnp.arange(F) // FD
    bm = (col[:, None] == col[None, :]).astype(jnp.float32)
    pmat = (jnp.arange(FD)[:, None] == col[None, :]).astype(jnp.float32)
    znum = jnp.zeros((CH, F), jnp.float32)

    hn = _tc_init(x, weight, bias.reshape(1, F), bm)
    hd = hn
    for _ in range(ITERS):
        num2, s2 = _sc_edge(hd, hn, src, dst, znum)
        s2r = s2.reshape(NC, NG * K, FD)[:, :N, :]
        hd = _tc_node(num2, s2r, hn, pmat, bm)
    return hd


# speedup vs baseline: 1.2841x; 1.2841x over previous
import jax
import jax.numpy as jnp
from jax import lax
from jax.experimental import pallas as pl
from jax.experimental.pallas import tpu as pltpu
from jax.experimental.pallas import tpu_sc as plsc

N = 10000
E = 320000
F = 128
K = 8
FD = 16
ITERS = 4

NC = 2
NS = 16
CH = 40
CHUNKS = E // CH
CPC = CHUNKS // NC
NJ = CPC // NS
RPT = 624
SP = 48
NSP = RPT // SP
TBASE = RPT * NS
NG = 1280
GPT = NG // NS

BL = 1000

_GDN = lax.GatherDimensionNumbers(
    offset_dims=(), collapsed_slice_dims=(0,), start_index_map=(0,))


def _perm(v, idx):
    return lax.gather(v, idx[:, None], _GDN, slice_sizes=(1,),
                      mode=lax.GatherScatterMode.PROMISE_IN_BOUNDS)


def _tc_init_body(x_ref, w_ref, b_ref, bm_ref, o_ref):
    h = jnp.dot(x_ref[...], w_ref[...], preferred_element_type=jnp.float32)
    h = h + b_ref[...]
    h = jnp.where(h >= 0.0, h, 0.01 * h)
    cs = jnp.dot(h * h, bm_ref[...], preferred_element_type=jnp.float32)
    o_ref[...] = h * lax.rsqrt(cs)


_tc_init = pl.pallas_call(
    _tc_init_body,
    grid=(N // BL,),
    in_specs=[
        pl.BlockSpec((BL, F), lambda i: (i, 0)),
        pl.BlockSpec((F, F), lambda i: (0, 0)),
        pl.BlockSpec((1, F), lambda i: (0, 0)),
        pl.BlockSpec((F, F), lambda i: (0, 0)),
    ],
    out_specs=pl.BlockSpec((BL, F), lambda i: (i, 0)),
    out_shape=jax.ShapeDtypeStruct((N, F), jnp.float32),
)


def _tc_node_body(num_ref, s_ref, hn_ref, p_ref, bm_ref, o_ref):
    nsum = num_ref[0] + num_ref[1]
    ssum = s_ref[0] + s_ref[1]
    sb = jnp.dot(ssum, p_ref[...], preferred_element_type=jnp.float32)
    sb = jnp.where(sb > 0.0, sb, 1.0)
    na = nsum / sb + hn_ref[...]
    cs = jnp.dot(na * na, bm_ref[...], preferred_element_type=jnp.float32)
    o_ref[...] = na * lax.rsqrt(cs)


_tc_node = pl.pallas_call(
    _tc_node_body,
    grid=(N // BL,),
    in_specs=[
        pl.BlockSpec((NC, BL, F), lambda i: (0, i, 0)),
        pl.BlockSpec((NC, BL, FD), lambda i: (0, i, 0)),
        pl.BlockSpec((BL, F), lambda i: (i, 0)),
        pl.BlockSpec((FD, F), lambda i: (0, 0)),
        pl.BlockSpec((F, F), lambda i: (0, 0)),
    ],
    out_specs=pl.BlockSpec((BL, F), lambda i: (i, 0)),
    out_shape=jax.ShapeDtypeStruct((N, F), jnp.float32),
)


def _sc_edge_body(hd, hn, src, dst, znum, num2, s2,
                  num_s, s_g,
                  isrcp0, idst0, hd0, hn0, ssc0, isc0, ix80,
                  isrcp1, idst1, hd1, hn1, ssc1, isc1, ix81,
                  semI0, semA0, semB0, semC0, semD0,
                  semI1, semA1, semB1, semC1, semD1):
    core = lax.axis_index("c")
    sub = lax.axis_index("s")
    r0 = sub * RPT
    g0 = sub * GPT

    pltpu.sync_copy(znum.at[pl.ds(0, CH)], ssc0)

    def zpiece(t, czp):
        pltpu.sync_copy(ssc0.at[pl.ds(0, CH)], num_s.at[pl.ds(r0 + CH * t, CH)])
        return czp

    lax.fori_loop(0, RPT // CH, zpiece, 0)
    pltpu.sync_copy(ssc0.at[pl.ds(0, 24)], num_s.at[pl.ds(r0 + (RPT // CH) * CH, 24)])
    pltpu.sync_copy(ssc0.at[pl.ds(0, FD)], num_s.at[pl.ds(TBASE, FD)])
    pltpu.sync_copy(ssc0, s_g.at[pl.ds(g0, CH)])
    pltpu.sync_copy(ssc0, s_g.at[pl.ds(g0 + CH, CH)])
    plsc.subcore_barrier()

    iot = lax.iota(jnp.int32, FD)
    perms = [jnp.bitwise_xor(iot, sh) for sh in (8, 4, 2, 1)]
    zrow = jnp.zeros((FD,), jnp.float32)
    cbase = core * CPC + sub
    clast = (core + 1) * CPC - NS + sub
    SETS = ((isrcp0, idst0, hd0, hn0, ssc0, isc0, ix80, semI0, semA0, semB0, semC0, semD0),
            (isrcp1, idst1, hd1, hn1, ssc1, isc1, ix81, semI1, semA1, semB1, semC1, semD1))

    def fire_idx(s, cc):
        isrcp, idst = s[0], s[1]
        eb = cc * CH
        pltpu.async_copy(src.at[pl.ds(eb, CH)], isrcp.at[pl.ds(0, CH)], s[7])
        pltpu.async_copy(dst.at[pl.ds(eb, CH)], idst, s[7])

    def wait_idx(s, cc):
        isrcp, idst = s[0], s[1]
        eb = cc * CH
        pltpu.make_async_copy(src.at[pl.ds(eb, CH)], isrcp.at[pl.ds(0, CH)], s[7]).wait()
        pltpu.make_async_copy(dst.at[pl.ds(eb, CH)], idst, s[7]).wait()

    def fire_gather(s, cc):
        wait_idx(s, cc)
        pltpu.async_copy(hd.at[s[0].at[pl.ds(0, CH)]], s[2], s[8])
        pltpu.async_copy(hn.at[s[1]], s[3], s[9])

    def compute(s):
        isrcp, idst, hd_b, hn_b, ssc, isc, ix8 = s[:7]
        pltpu.make_async_copy(hd.at[isrcp.at[pl.ds(0, CH)]], hd_b, s[8]).wait()
        pltpu.make_async_copy(hn.at[idst], hn_b, s[9]).wait()

        def edge(i2, carry2):
            for half in range(2):
                i = i2 * 2 + half
                exrow = zrow
                for k in range(K):
                    a = hd_b[i, pl.ds(k * FD, FD)]
                    b = hn_b[i, pl.ds(k * FD, FD)]
                    p = a * b
                    for pm in perms:
                        p = p + _perm(p, pm)
                    ek = jnp.exp(p)
                    hn_b[i, pl.ds(k * FD, FD)] = b * ek
                    exrow = jnp.where(iot == k, ek, exrow)
                sv = isrcp[pl.ds(i, FD)]
                boff = jnp.bitwise_and(sv[0], 7)
                for blk in range(8):
                    ssc[i, pl.ds(blk * FD, FD)] = jnp.where(boff == blk, exrow, zrow)
            return carry2

        lax.fori_loop(0, CH // 2, edge, 0)
        for off in (0, FD, CH - FD):
            sl = isrcp[pl.ds(off, FD)]
            isc[pl.ds(off, FD)] = sl
            ix8[pl.ds(off, FD)] = lax.shift_right_logical(sl, 3)
        pltpu.sync_copy(hn_b, num_s.at[isc], add=True)
        pltpu.sync_copy(ssc, s_g.at[ix8], add=True)

    s0, s1 = SETS
    fire_idx(s0, cbase)
    fire_gather(s0, cbase)
    fire_idx(s1, cbase + NS)

    def pair(j2, carry):
        j = 2 * j2
        fire_gather(s1, jnp.minimum(cbase + NS * (j + 1), clast))
        compute(s0)
        fire_idx(s0, jnp.minimum(cbase + NS * (j + 2), clast))
        fire_gather(s0, jnp.minimum(cbase + NS * (j + 2), clast))
        compute(s1)
        fire_idx(s1, jnp.minimum(cbase + NS * (j + 3), clast))
        return carry

    lax.fori_loop(0, NJ // 2, pair, 0)
    pltpu.make_async_copy(hd.at[isrcp0.at[pl.ds(0, CH)]], hd0, semA0).wait()
    pltpu.make_async_copy(hn.at[idst0], hn0, semB0).wait()
    wait_idx(s1, clast)
    plsc.subcore_barrier()

    def opiece(t, cop):
        ro = r0 + CH * t
        pltpu.sync_copy(num_s.at[pl.ds(ro, CH)], hd0.at[pl.ds(0, CH)])
        pltpu.sync_copy(hd0.at[pl.ds(0, CH)], num2.at[core, pl.ds(ro, CH)])
        return cop

    lax.fori_loop(0, RPT // CH, opiece, 0)
    rt = r0 + (RPT // CH) * CH
    pltpu.sync_copy(num_s.at[pl.ds(rt, 24)], hd0.at[pl.ds(0, 24)])
    pltpu.sync_copy(hd0.at[pl.ds(0, 24)], num2.at[core, pl.ds(rt, 24)])
    pltpu.sync_copy(num_s.at[pl.ds(TBASE, FD)], hd0.at[pl.ds(0, FD)])
    pltpu.sync_copy(hd0.at[pl.ds(0, FD)], num2.at[core, pl.ds(TBASE, FD)])
    pltpu.sync_copy(s_g.at[pl.ds(g0, CH)], hd0.at[pl.ds(0, CH)])
    pltpu.sync_copy(hd0.at[pl.ds(0, CH)], s2.at[core, pl.ds(g0, CH)])
    pltpu.sync_copy(s_g.at[pl.ds(g0 + CH, CH)], hd0.at[pl.ds(0, CH)])
    pltpu.sync_copy(hd0.at[pl.ds(0, CH)], s2.at[core, pl.ds(g0 + CH, CH)])


_sc_edge = pl.kernel(
    _sc_edge_body,
    out_type=(
        jax.ShapeDtypeStruct((NC, N, F), jnp.float32),
        jax.ShapeDtypeStruct((NC, NG, F), jnp.float32),
    ),
    mesh=plsc.VectorSubcoreMesh(
        core_axis_name="c", subcore_axis_name="s", num_cores=NC, num_subcores=NS
    ),
    scratch_types=[
        pltpu.VMEM_SHARED((N, F), jnp.float32),
        pltpu.VMEM_SHARED((NG, F), jnp.float32),
        pltpu.VMEM((CH + FD,), jnp.int32),
        pltpu.VMEM((CH,), jnp.int32),
        pltpu.VMEM((CH, F), jnp.float32),
        pltpu.VMEM((CH, F), jnp.float32),
        pltpu.VMEM((CH, F), jnp.float32),
        pltpu.VMEM((CH,), jnp.int32),
        pltpu.VMEM((CH,), jnp.int32),
        pltpu.VMEM((CH + FD,), jnp.int32),
        pltpu.VMEM((CH,), jnp.int32),
        pltpu.VMEM((CH, F), jnp.float32),
        pltpu.VMEM((CH, F), jnp.float32),
        pltpu.VMEM((CH, F), jnp.float32),
        pltpu.VMEM((CH,), jnp.int32),
        pltpu.VMEM((CH,), jnp.int32),
        pltpu.SemaphoreType.DMA,
        pltpu.SemaphoreType.DMA,
        pltpu.SemaphoreType.DMA,
        pltpu.SemaphoreType.DMA,
        pltpu.SemaphoreType.DMA,
        pltpu.SemaphoreType.DMA,
        pltpu.SemaphoreType.DMA,
        pltpu.SemaphoreType.DMA,
        pltpu.SemaphoreType.DMA,
        pltpu.SemaphoreType.DMA,
    ],
)


@jax.jit
def kernel(x, edge_index, weight, bias):
    src = edge_index[0]
    dst = edge_index[1]
    col = jnp.arange(F) // FD
    bm = (col[:, None] == col[None, :]).astype(jnp.float32)
    pmat = (jnp.arange(FD)[:, None] == col[None, :]).astype(jnp.float32)
    znum = jnp.zeros((CH, F), jnp.float32)

    hn = _tc_init(x, weight, bias.reshape(1, F), bm)
    hd = hn
    for _ in range(ITERS):
        num2, s2 = _sc_edge(hd, hn, src, dst, znum)
        s2r = s2.reshape(NC, NG * K, FD)[:, :N, :]
        hd = _tc_node(num2, s2r, hn, pmat, bm)
    return hd


# speedup vs baseline: 1.5442x; 1.2025x over previous
import jax
import jax.numpy as jnp
from jax import lax
from jax.experimental import pallas as pl
from jax.experimental.pallas import tpu as pltpu
from jax.experimental.pallas import tpu_sc as plsc

N = 10000
E = 320000
F = 128
K = 8
FD = 16
ITERS = 4

NC = 2
NS = 16
CH = 40
CHUNKS = E // CH
CPC = CHUNKS // NC
NJ = CPC // NS
RPT = 624
SP = 48
NSP = RPT // SP
TBASE = RPT * NS
NG = 1280
GPT = NG // NS

BL = 1000

_GDN = lax.GatherDimensionNumbers(
    offset_dims=(), collapsed_slice_dims=(0,), start_index_map=(0,))


def _perm(v, idx):
    return lax.gather(v, idx[:, None], _GDN, slice_sizes=(1,),
                      mode=lax.GatherScatterMode.PROMISE_IN_BOUNDS)


def _tc_init_body(x_ref, w_ref, b_ref, bm_ref, o_ref):
    h = jnp.dot(x_ref[...], w_ref[...], preferred_element_type=jnp.float32)
    h = h + b_ref[...]
    h = jnp.where(h >= 0.0, h, 0.01 * h)
    cs = jnp.dot(h * h, bm_ref[...], preferred_element_type=jnp.float32)
    o_ref[...] = h * lax.rsqrt(cs)


_tc_init = pl.pallas_call(
    _tc_init_body,
    grid=(N // BL,),
    in_specs=[
        pl.BlockSpec((BL, F), lambda i: (i, 0)),
        pl.BlockSpec((F, F), lambda i: (0, 0)),
        pl.BlockSpec((1, F), lambda i: (0, 0)),
        pl.BlockSpec((F, F), lambda i: (0, 0)),
    ],
    out_specs=pl.BlockSpec((BL, F), lambda i: (i, 0)),
    out_shape=jax.ShapeDtypeStruct((N, F), jnp.float32),
)


def _tc_node_body(num_ref, s_ref, hn_ref, p_ref, bm_ref, o_ref):
    nsum = num_ref[0] + num_ref[1]
    ssum = s_ref[0] + s_ref[1]
    sb = jnp.dot(ssum, p_ref[...], preferred_element_type=jnp.float32)
    sb = jnp.where(sb > 0.0, sb, 1.0)
    na = nsum / sb + hn_ref[...]
    cs = jnp.dot(na * na, bm_ref[...], preferred_element_type=jnp.float32)
    o_ref[...] = na * lax.rsqrt(cs)


_tc_node = pl.pallas_call(
    _tc_node_body,
    grid=(N // BL,),
    in_specs=[
        pl.BlockSpec((NC, BL, F), lambda i: (0, i, 0)),
        pl.BlockSpec((NC, BL, FD), lambda i: (0, i, 0)),
        pl.BlockSpec((BL, F), lambda i: (i, 0)),
        pl.BlockSpec((FD, F), lambda i: (0, 0)),
        pl.BlockSpec((F, F), lambda i: (0, 0)),
    ],
    out_specs=pl.BlockSpec((BL, F), lambda i: (i, 0)),
    out_shape=jax.ShapeDtypeStruct((N, F), jnp.float32),
)


def _sc_edge_body(hd, hn, src, dst, znum, num2, s2,
                  num_s, s_g,
                  isrcp0, idst0, hd0, hn0, ssc0, isc0, ix80,
                  isrcp1, idst1, hd1, hn1, ssc1, isc1, ix81,
                  semI0, semA0, semB0, semC0, semD0,
                  semI1, semA1, semB1, semC1, semD1):
    core = lax.axis_index("c")
    sub = lax.axis_index("s")
    r0 = sub * RPT
    g0 = sub * GPT

    pltpu.sync_copy(znum.at[pl.ds(0, CH)], ssc0)

    def zpiece(t, czp):
        pltpu.sync_copy(ssc0.at[pl.ds(0, CH)], num_s.at[pl.ds(r0 + CH * t, CH)])
        return czp

    lax.fori_loop(0, RPT // CH, zpiece, 0)
    pltpu.sync_copy(ssc0.at[pl.ds(0, 24)], num_s.at[pl.ds(r0 + (RPT // CH) * CH, 24)])
    pltpu.sync_copy(ssc0.at[pl.ds(0, FD)], num_s.at[pl.ds(TBASE, FD)])
    pltpu.sync_copy(ssc0, s_g.at[pl.ds(g0, CH)])
    pltpu.sync_copy(ssc0, s_g.at[pl.ds(g0 + CH, CH)])
    plsc.subcore_barrier()

    iot = lax.iota(jnp.int32, FD)
    perms = [jnp.bitwise_xor(iot, sh) for sh in (8, 4, 2, 1)]
    zrow = jnp.zeros((FD,), jnp.float32)
    cbase = core * CPC + sub
    clast = (core + 1) * CPC - NS + sub
    SETS = ((isrcp0, idst0, hd0, hn0, ssc0, isc0, ix80, semI0, semA0, semB0, semC0, semD0),
            (isrcp1, idst1, hd1, hn1, ssc1, isc1, ix81, semI1, semA1, semB1, semC1, semD1))

    def fire_idx(s, cc):
        isrcp, idst = s[0], s[1]
        eb = cc * CH
        pltpu.async_copy(src.at[pl.ds(eb, CH)], isrcp.at[pl.ds(0, CH)], s[7])
        pltpu.async_copy(dst.at[pl.ds(eb, CH)], idst, s[7])

    def wait_idx(s, cc):
        isrcp, idst = s[0], s[1]
        eb = cc * CH
        pltpu.make_async_copy(src.at[pl.ds(eb, CH)], isrcp.at[pl.ds(0, CH)], s[7]).wait()
        pltpu.make_async_copy(dst.at[pl.ds(eb, CH)], idst, s[7]).wait()

    def wait_sc(s):
        pltpu.make_async_copy(s[3], num_s.at[s[5]], s[10]).wait()
        pltpu.make_async_copy(s[4], s_g.at[s[6]], s[11]).wait()

    def fire_gather(s, cc, first=False):
        wait_idx(s, cc)
        if not first:
            wait_sc(s)
        pltpu.async_copy(hd.at[s[0].at[pl.ds(0, CH)]], s[2], s[8])
        pltpu.async_copy(hn.at[s[1]], s[3], s[9])

    def compute(s):
        isrcp, idst, hd_b, hn_b, ssc, isc, ix8 = s[:7]
        pltpu.make_async_copy(hd.at[isrcp.at[pl.ds(0, CH)]], hd_b, s[8]).wait()
        pltpu.make_async_copy(hn.at[idst], hn_b, s[9]).wait()

        def edge(i2, carry2):
            for half in range(2):
                i = i2 * 2 + half
                exrow = zrow
                for k in range(K):
                    a = hd_b[i, pl.ds(k * FD, FD)]
                    b = hn_b[i, pl.ds(k * FD, FD)]
                    p = a * b
                    for pm in perms:
                        p = p + _perm(p, pm)
                    ek = jnp.exp(p)
                    hn_b[i, pl.ds(k * FD, FD)] = b * ek
                    exrow = jnp.where(iot == k, ek, exrow)
                sv = isrcp[pl.ds(i, FD)]
                boff = jnp.bitwise_and(sv[0], 7)
                for blk in range(8):
                    ssc[i, pl.ds(blk * FD, FD)] = jnp.where(boff == blk, exrow, zrow)
            return carry2

        lax.fori_loop(0, CH // 2, edge, 0)
        for off in (0, FD, CH - FD):
            sl = isrcp[pl.ds(off, FD)]
            isc[pl.ds(off, FD)] = sl
            ix8[pl.ds(off, FD)] = lax.shift_right_logical(sl, 3)
        pltpu.async_copy(hn_b, num_s.at[isc], s[10], add=True)
        pltpu.async_copy(ssc, s_g.at[ix8], s[11], add=True)

    s0, s1 = SETS
    fire_idx(s0, cbase)
    fire_gather(s0, cbase, first=True)
    fire_idx(s1, cbase + NS)

    def pair(j2, carry, first=False):
        j = 2 * j2
        fire_gather(s1, jnp.minimum(cbase + NS * (j + 1), clast), first=first)
        compute(s0)
        fire_idx(s0, jnp.minimum(cbase + NS * (j + 2), clast))
        fire_gather(s0, jnp.minimum(cbase + NS * (j + 2), clast))
        compute(s1)
        fire_idx(s1, jnp.minimum(cbase + NS * (j + 3), clast))
        return carry

    pair(0, 0, first=True)
    lax.fori_loop(1, NJ // 2, pair, 0)
    pltpu.make_async_copy(hd.at[isrcp0.at[pl.ds(0, CH)]], hd0, semA0).wait()
    pltpu.make_async_copy(hn.at[idst0], hn0, semB0).wait()
    wait_idx(s1, clast)
    wait_sc(s1)
    plsc.subcore_barrier()

    def opiece(t, cop):
        ro = r0 + CH * t
        pltpu.sync_copy(num_s.at[pl.ds(ro, CH)], hd0.at[pl.ds(0, CH)])
        pltpu.sync_copy(hd0.at[pl.ds(0, CH)], num2.at[core, pl.ds(ro, CH)])
        return cop

    lax.fori_loop(0, RPT // CH, opiece, 0)
    rt = r0 + (RPT // CH) * CH
    pltpu.sync_copy(num_s.at[pl.ds(rt, 24)], hd0.at[pl.ds(0, 24)])
    pltpu.sync_copy(hd0.at[pl.ds(0, 24)], num2.at[core, pl.ds(rt, 24)])
    pltpu.sync_copy(num_s.at[pl.ds(TBASE, FD)], hd0.at[pl.ds(0, FD)])
    pltpu.sync_copy(hd0.at[pl.ds(0, FD)], num2.at[core, pl.ds(TBASE, FD)])
    pltpu.sync_copy(s_g.at[pl.ds(g0, CH)], hd0.at[pl.ds(0, CH)])
    pltpu.sync_copy(hd0.at[pl.ds(0, CH)], s2.at[core, pl.ds(g0, CH)])
    pltpu.sync_copy(s_g.at[pl.ds(g0 + CH, CH)], hd0.at[pl.ds(0, CH)])
    pltpu.sync_copy(hd0.at[pl.ds(0, CH)], s2.at[core, pl.ds(g0 + CH, CH)])


_sc_edge = pl.kernel(
    _sc_edge_body,
    out_type=(
        jax.ShapeDtypeStruct((NC, N, F), jnp.float32),
        jax.ShapeDtypeStruct((NC, NG, F), jnp.float32),
    ),
    mesh=plsc.VectorSubcoreMesh(
        core_axis_name="c", subcore_axis_name="s", num_cores=NC, num_subcores=NS
    ),
    scratch_types=[
        pltpu.VMEM_SHARED((N, F), jnp.float32),
        pltpu.VMEM_SHARED((NG, F), jnp.float32),
        pltpu.VMEM((CH + FD,), jnp.int32),
        pltpu.VMEM((CH,), jnp.int32),
        pltpu.VMEM((CH, F), jnp.float32),
        pltpu.VMEM((CH, F), jnp.float32),
        pltpu.VMEM((CH, F), jnp.float32),
        pltpu.VMEM((CH,), jnp.int32),
        pltpu.VMEM((CH,), jnp.int32),
        pltpu.VMEM((CH + FD,), jnp.int32),
        pltpu.VMEM((CH,), jnp.int32),
        pltpu.VMEM((CH, F), jnp.float32),
        pltpu.VMEM((CH, F), jnp.float32),
        pltpu.VMEM((CH, F), jnp.float32),
        pltpu.VMEM((CH,), jnp.int32),
        pltpu.VMEM((CH,), jnp.int32),
        pltpu.SemaphoreType.DMA,
        pltpu.SemaphoreType.DMA,
        pltpu.SemaphoreType.DMA,
        pltpu.SemaphoreType.DMA,
        pltpu.SemaphoreType.DMA,
        pltpu.SemaphoreType.DMA,
        pltpu.SemaphoreType.DMA,
        pltpu.SemaphoreType.DMA,
        pltpu.SemaphoreType.DMA,
        pltpu.SemaphoreType.DMA,
    ],
)


@jax.jit
def kernel(x, edge_index, weight, bias):
    src = edge_index[0]
    dst = edge_index[1]
    col = jnp.arange(F) // FD
    bm = (col[:, None] == col[None, :]).astype(jnp.float32)
    pmat = (jnp.arange(FD)[:, None] == col[None, :]).astype(jnp.float32)
    znum = jnp.zeros((CH, F), jnp.float32)

    hn = _tc_init(x, weight, bias.reshape(1, F), bm)
    hd = hn
    for _ in range(ITERS):
        num2, s2 = _sc_edge(hd, hn, src, dst, znum)
        s2r = s2.reshape(NC, NG * K, FD)[:, :N, :]
        hd = _tc_node(num2, s2r, hn, pmat, bm)
    return hd
